# baseline (speedup 1.0000x reference)
import jax
import jax.numpy as jnp
from jax.experimental import pallas as pl
from jax.experimental.pallas import tpu as pltpu

N = 10000
BI = 200
NI = N // BI
TOT = 2 * NI
NRING = 4


def _vof(t):
    return jnp.where(t < NI, t, TOT - 1 - t)


def _gcn_kernel(y_ref, w1_ref, b1_ref, w2_ref, b2_ref, adj_hbm,
                h_ref, out_ref, s1_ref, s2_ref, ring_ref, sems):
    p = pl.program_id(0)
    i = pl.program_id(1)
    s = p * NI + i

    def start(v):
        sl = jax.lax.rem(v, NRING)
        pltpu.make_async_copy(
            adj_hbm.at[pl.ds(v * BI, BI), :],
            ring_ref.at[sl],
            sems.at[sl],
        ).start()

    def wait(v):
        sl = jax.lax.rem(v, NRING)
        pltpu.make_async_copy(
            adj_hbm.at[pl.ds(v * BI, BI), :],
            ring_ref.at[sl],
            sems.at[sl],
        ).wait()

    def needs_fetch(t):
        return (t < NI) | (_vof(t) <= NI - 1 - NRING)

    @pl.when(s == 0)
    def _():
        start(_vof(0))
        start(_vof(1))
        start(_vof(2))

    @pl.when((s + 3 < TOT) & needs_fetch(s + 3))
    def _():
        start(_vof(s + 3))

    @pl.when((p == 0) & (i == 0))
    def _():
        s1_ref[...] = jnp.dot(
            y_ref[...], w1_ref[...], preferred_element_type=jnp.float32
        )

    @pl.when((p == 1) & (i == 0))
    def _():
        s2_ref[...] = jnp.dot(
            h_ref[...], w2_ref[...], preferred_element_type=jnp.float32
        )

    @pl.when(needs_fetch(s))
    def _():
        wait(_vof(s))

    v = _vof(s)
    ab = ring_ref[jax.lax.rem(v, NRING)]

    @pl.when(p == 0)
    def _():
        t = jnp.dot(ab, s1_ref[...],
                    preferred_element_type=jnp.float32) + b1_ref[...]
        h_ref[pl.ds(i * BI, BI), :] = jnp.where(t >= 0, t, 0.01 * t)

    @pl.when(p == 1)
    def _():
        t = jnp.dot(ab, s2_ref[...],
                    preferred_element_type=jnp.float32) + b2_ref[...]
        m = jnp.max(t, axis=1, keepdims=True)
        e = jnp.exp(t - m)
        out_ref[...] = e / jnp.sum(e, axis=1, keepdims=True)


def kernel(y, adj, W1, b1, W2, b2):
    nfeat = W1.shape[0]
    nhid = W1.shape[1]
    nclass = W2.shape[1]
    h, out = pl.pallas_call(
        _gcn_kernel,
        grid=(2, NI),
        in_specs=[
            pl.BlockSpec((N, nfeat), lambda p, i: (0, 0),
                         pipeline_mode=pl.Buffered(buffer_count=1)),
            pl.BlockSpec((nfeat, nhid), lambda p, i: (0, 0),
                         pipeline_mode=pl.Buffered(buffer_count=1)),
            pl.BlockSpec((1, nhid), lambda p, i: (0, 0),
                         pipeline_mode=pl.Buffered(buffer_count=1)),
            pl.BlockSpec((nhid, nclass), lambda p, i: (0, 0),
                         pipeline_mode=pl.Buffered(buffer_count=1)),
            pl.BlockSpec((1, nclass), lambda p, i: (0, 0),
                         pipeline_mode=pl.Buffered(buffer_count=1)),
            pl.BlockSpec(memory_space=pltpu.MemorySpace.HBM),
        ],
        out_specs=[
            pl.BlockSpec((N, nhid), lambda p, i: (0, 0),
                         pipeline_mode=pl.Buffered(buffer_count=1)),
            pl.BlockSpec((BI, nclass),
                         lambda p, i: (jnp.where(p == 0, NI - 1, NI - 1 - i),
                                       0)),
        ],
        out_shape=[
            jax.ShapeDtypeStruct((N, nhid), jnp.float32),
            jax.ShapeDtypeStruct((N, nclass), jnp.float32),
        ],
        scratch_shapes=[
            pltpu.VMEM((N, nhid), jnp.float32),
            pltpu.VMEM((N, nclass), jnp.float32),
            pltpu.VMEM((NRING, BI, N), jnp.float32),
            pltpu.SemaphoreType.DMA((NRING,)),
        ],
        compiler_params=pltpu.CompilerParams(
            vmem_limit_bytes=64 * 1024 * 1024,
        ),
    )(y, W1, b1.reshape(1, nhid), W2, b2.reshape(1, nclass), adj)
    return (out, h)


# speedup vs baseline: 1.0110x; 1.0110x over previous
import jax
import jax.numpy as jnp
from jax.experimental import pallas as pl
from jax.experimental.pallas import tpu as pltpu

N = 10000
BI = 200
NI = N // BI
TOT = 2 * NI
NRING = 4


def _vof(t):
    return jnp.where(t < NI, t, TOT - 1 - t)


def _gcn_kernel(y_ref, w1_ref, b1_ref, w2_ref, b2_ref, adj_hbm,
                h_ref, out_ref, s1_ref, s2_ref, ring_ref, sems):
    p = pl.program_id(0)
    i = pl.program_id(1)
    s = p * NI + i

    def start(v):
        sl = jax.lax.rem(v, NRING)
        pltpu.make_async_copy(
            adj_hbm.at[pl.ds(v * BI, BI), :],
            ring_ref.at[sl],
            sems.at[sl],
        ).start()

    def wait(v):
        sl = jax.lax.rem(v, NRING)
        pltpu.make_async_copy(
            adj_hbm.at[pl.ds(v * BI, BI), :],
            ring_ref.at[sl],
            sems.at[sl],
        ).wait()

    def needs_fetch(t):
        return (t < NI) | (_vof(t) <= NI - 1 - NRING)

    @pl.when(s == 0)
    def _():
        start(_vof(0))
        start(_vof(1))

    @pl.when((s + 2 < TOT) & needs_fetch(s + 2))
    def _():
        start(_vof(s + 2))

    @pl.when((p == 0) & (i == 0))
    def _():
        s1_ref[...] = jnp.dot(
            y_ref[...], w1_ref[...], preferred_element_type=jnp.float32
        )

    @pl.when((p == 1) & (i == 0))
    def _():
        s2_ref[...] = jnp.dot(
            h_ref[...], w2_ref[...], preferred_element_type=jnp.float32
        )

    @pl.when(needs_fetch(s))
    def _():
        wait(_vof(s))

    v = _vof(s)
    ab = ring_ref[jax.lax.rem(v, NRING)]

    @pl.when(p == 0)
    def _():
        t = jnp.dot(ab, s1_ref[...],
                    preferred_element_type=jnp.float32) + b1_ref[...]
        h_ref[pl.ds(i * BI, BI), :] = jnp.where(t >= 0, t, 0.01 * t)

    @pl.when(p == 1)
    def _():
        t = jnp.dot(ab, s2_ref[...],
                    preferred_element_type=jnp.float32) + b2_ref[...]
        m = jnp.max(t, axis=1, keepdims=True)
        e = jnp.exp(t - m)
        out_ref[...] = e / jnp.sum(e, axis=1, keepdims=True)


def kernel(y, adj, W1, b1, W2, b2):
    nfeat = W1.shape[0]
    nhid = W1.shape[1]
    nclass = W2.shape[1]
    h, out = pl.pallas_call(
        _gcn_kernel,
        grid=(2, NI),
        in_specs=[
            pl.BlockSpec((N, nfeat), lambda p, i: (0, 0),
                         pipeline_mode=pl.Buffered(buffer_count=1)),
            pl.BlockSpec((nfeat, nhid), lambda p, i: (0, 0),
                         pipeline_mode=pl.Buffered(buffer_count=1)),
            pl.BlockSpec((1, nhid), lambda p, i: (0, 0),
                         pipeline_mode=pl.Buffered(buffer_count=1)),
            pl.BlockSpec((nhid, nclass), lambda p, i: (0, 0),
                         pipeline_mode=pl.Buffered(buffer_count=1)),
            pl.BlockSpec((1, nclass), lambda p, i: (0, 0),
                         pipeline_mode=pl.Buffered(buffer_count=1)),
            pl.BlockSpec(memory_space=pltpu.MemorySpace.HBM),
        ],
        out_specs=[
            pl.BlockSpec((N, nhid), lambda p, i: (0, 0),
                         pipeline_mode=pl.Buffered(buffer_count=1)),
            pl.BlockSpec((BI, nclass),
                         lambda p, i: (jnp.where(p == 0, NI - 1, NI - 1 - i),
                                       0)),
        ],
        out_shape=[
            jax.ShapeDtypeStruct((N, nhid), jnp.float32),
            jax.ShapeDtypeStruct((N, nclass), jnp.float32),
        ],
        scratch_shapes=[
            pltpu.VMEM((N, nhid), jnp.float32),
            pltpu.VMEM((N, nclass), jnp.float32),
            pltpu.VMEM((NRING, BI, N), jnp.float32),
            pltpu.SemaphoreType.DMA((NRING,)),
        ],
        compiler_params=pltpu.CompilerParams(
            vmem_limit_bytes=64 * 1024 * 1024,
        ),
    )(y, W1, b1.reshape(1, nhid), W2, b2.reshape(1, nclass), adj)
    return (out, h)
